# baseline (speedup 1.0000x reference)
import functools

import jax
import jax.numpy as jnp
from jax import lax
from jax.experimental import pallas as pl
from jax.experimental.pallas import tpu as pltpu
from jax.experimental.pallas import tpu_sc as plsc

N = 1048576
NUM_CORES = 2
NUM_SUBCORES = 16
NUM_WORKERS = NUM_CORES * NUM_SUBCORES
CHUNK = N // NUM_WORKERS
LANES = 16
BINS = 64
OFFSET = 32
BANKS = 4
BANK_WORDS = BINS * LANES
HIST_WORDS = BANKS * BANK_WORDS
ITERS = CHUNK // (LANES * BANKS)
MAGIC = 12582912.0


def _sc_hist_body(t_hbm, o_hbm, cnt_out, ssq_out, t_v, o_v, cnt_v, ssq_v, sem_t, sem_o):
    wid = lax.axis_index("s") * NUM_CORES + lax.axis_index("c")
    base = wid * CHUNK

    cp_t = pltpu.make_async_copy(t_hbm.at[pl.ds(base, CHUNK)], t_v, sem_t)
    cp_o = pltpu.make_async_copy(o_hbm.at[pl.ds(base, CHUNK)], o_v, sem_o)
    cp_t.start()
    cp_o.start()

    zeros = jnp.zeros((LANES,), jnp.float32)
    for j in range(HIST_WORDS // LANES):
        cnt_v[pl.ds(j * LANES, LANES)] = zeros
        ssq_v[pl.ds(j * LANES, LANES)] = zeros

    lane = lax.iota(jnp.int32, LANES)
    lane_f = lane.astype(jnp.float32)
    ones = jnp.ones((LANES,), jnp.float32)
    addc = [lane_f + float(k * BANK_WORDS + OFFSET * LANES) for k in range(BANKS)]
    lo = [lane + jnp.int32(k * BANK_WORDS) for k in range(BANKS)]
    hi = [lane + jnp.int32(k * BANK_WORDS + (BINS - 1) * LANES) for k in range(BANKS)]

    cp_t.wait()
    cp_o.wait()

    def body(i, carry):
        off = i * (LANES * BANKS)
        for k in range(BANKS):
            s = off + k * LANES
            t = t_v[pl.ds(s, LANES)]
            o = o_v[pl.ds(s, LANES)]
            d = t - o
            d2 = d * d
            r = (t + MAGIC) - MAGIC
            flat = (r * float(LANES) + addc[k]).astype(jnp.int32)
            flat = jnp.minimum(jnp.maximum(flat, lo[k]), hi[k])
            plsc.addupdate_scatter(cnt_v, [flat], ones)
            plsc.addupdate_scatter(ssq_v, [flat], d2)
        return carry

    lax.fori_loop(0, ITERS, body, 0)

    pltpu.sync_copy(cnt_v, cnt_out.at[wid])
    pltpu.sync_copy(ssq_v, ssq_out.at[wid])


_sc_hist = functools.partial(
    pl.kernel,
    out_type=[
        jax.ShapeDtypeStruct((NUM_WORKERS, HIST_WORDS), jnp.float32),
        jax.ShapeDtypeStruct((NUM_WORKERS, HIST_WORDS), jnp.float32),
    ],
    mesh=plsc.VectorSubcoreMesh(core_axis_name="c", subcore_axis_name="s"),
    compiler_params=pltpu.CompilerParams(needs_layout_passes=False),
    scratch_types=[
        pltpu.VMEM((CHUNK,), jnp.float32),
        pltpu.VMEM((CHUNK,), jnp.float32),
        pltpu.VMEM((HIST_WORDS,), jnp.float32),
        pltpu.VMEM((HIST_WORDS,), jnp.float32),
        pltpu.SemaphoreType.DMA,
        pltpu.SemaphoreType.DMA,
    ],
)(_sc_hist_body)


def _tc_finish_body(cnt_ref, ssq_ref, out_ref):
    cnt = jnp.sum(cnt_ref[...], axis=(0, 1, 3))
    ssq = jnp.sum(ssq_ref[...], axis=(0, 1, 3))
    maxw = jnp.max(cnt)
    nonzero = cnt > 0.0
    safe = jnp.where(nonzero, cnt, 1.0)
    total = jnp.sum(jnp.where(nonzero, ssq / safe, 0.0))
    out_ref[0, 0] = maxw * total * (1.0 / N)


def kernel(target, output):
    t = target.reshape(N)
    o = output.reshape(N)
    cnt_p, ssq_p = _sc_hist(t, o)
    cnt_p = cnt_p.reshape(NUM_WORKERS, BANKS, BINS, LANES)
    ssq_p = ssq_p.reshape(NUM_WORKERS, BANKS, BINS, LANES)
    loss = pl.pallas_call(
        _tc_finish_body,
        out_shape=jax.ShapeDtypeStruct((1, 1), jnp.float32),
        out_specs=pl.BlockSpec(memory_space=pltpu.SMEM),
    )(cnt_p, ssq_p)
    return loss[0, 0]


# speedup vs baseline: 1.5818x; 1.5818x over previous
import functools

import jax
import jax.numpy as jnp
from jax import lax
from jax.experimental import pallas as pl
from jax.experimental.pallas import tpu as pltpu
from jax.experimental.pallas import tpu_sc as plsc

N = 1048576
NUM_CORES = 2
NUM_SUBCORES = 16
NUM_WORKERS = NUM_CORES * NUM_SUBCORES
CHUNK = N // NUM_WORKERS
LANES = 16
BINS = 64
OFFSET = 32
BANKS = 4
BANK_WORDS = BINS * LANES
HIST_WORDS = BANKS * BANK_WORDS
ITERS = CHUNK // (LANES * BANKS)
MAGIC = 12582912.0


def _sc_hist_body(t_hbm, o_hbm, cnt_out, ssq_out, t_v, o_v, cnt_v, ssq_v, sem_t, sem_o):
    wid = lax.axis_index("s") * NUM_CORES + lax.axis_index("c")
    base = wid * CHUNK

    cp_t = pltpu.make_async_copy(t_hbm.at[pl.ds(base, CHUNK)], t_v, sem_t)
    cp_o = pltpu.make_async_copy(o_hbm.at[pl.ds(base, CHUNK)], o_v, sem_o)
    cp_t.start()
    cp_o.start()

    zeros = jnp.zeros((LANES,), jnp.float32)
    for j in range(HIST_WORDS // LANES):
        cnt_v[pl.ds(j * LANES, LANES)] = zeros
        ssq_v[pl.ds(j * LANES, LANES)] = zeros

    lane = lax.iota(jnp.int32, LANES)
    lane_f = lane.astype(jnp.float32)
    ones = jnp.ones((LANES,), jnp.float32)
    addc = [lane_f + float(k * BANK_WORDS + OFFSET * LANES) for k in range(BANKS)]
    lo = [lane + jnp.int32(k * BANK_WORDS) for k in range(BANKS)]
    hi = [lane + jnp.int32(k * BANK_WORDS + (BINS - 1) * LANES) for k in range(BANKS)]

    cp_t.wait()
    cp_o.wait()

    @plsc.parallel_loop(0, ITERS, 1, unroll=2)
    def body(i):
        off = i * (LANES * BANKS)
        for k in range(BANKS):
            s = off + k * LANES
            t = t_v[pl.ds(s, LANES)]
            o = o_v[pl.ds(s, LANES)]
            d = t - o
            d2 = d * d
            r = (t + MAGIC) - MAGIC
            flat = (r * float(LANES) + addc[k]).astype(jnp.int32)
            flat = jnp.minimum(jnp.maximum(flat, lo[k]), hi[k])
            plsc.addupdate_scatter(cnt_v, [flat], ones)
            plsc.addupdate_scatter(ssq_v, [flat], d2)

    pltpu.sync_copy(cnt_v, cnt_out.at[wid])
    pltpu.sync_copy(ssq_v, ssq_out.at[wid])


_sc_hist = functools.partial(
    pl.kernel,
    out_type=[
        jax.ShapeDtypeStruct((NUM_WORKERS, HIST_WORDS), jnp.float32),
        jax.ShapeDtypeStruct((NUM_WORKERS, HIST_WORDS), jnp.float32),
    ],
    mesh=plsc.VectorSubcoreMesh(core_axis_name="c", subcore_axis_name="s"),
    compiler_params=pltpu.CompilerParams(needs_layout_passes=False),
    scratch_types=[
        pltpu.VMEM((CHUNK,), jnp.float32),
        pltpu.VMEM((CHUNK,), jnp.float32),
        pltpu.VMEM((HIST_WORDS,), jnp.float32),
        pltpu.VMEM((HIST_WORDS,), jnp.float32),
        pltpu.SemaphoreType.DMA,
        pltpu.SemaphoreType.DMA,
    ],
)(_sc_hist_body)


def _tc_finish_body(cnt_ref, ssq_ref, out_ref):
    cnt = jnp.sum(cnt_ref[...], axis=(0, 1, 3))
    ssq = jnp.sum(ssq_ref[...], axis=(0, 1, 3))
    maxw = jnp.max(cnt)
    nonzero = cnt > 0.0
    safe = jnp.where(nonzero, cnt, 1.0)
    total = jnp.sum(jnp.where(nonzero, ssq / safe, 0.0))
    out_ref[0, 0] = maxw * total * (1.0 / N)


def kernel(target, output):
    t = target.reshape(N)
    o = output.reshape(N)
    cnt_p, ssq_p = _sc_hist(t, o)
    cnt_p = cnt_p.reshape(NUM_WORKERS, BANKS, BINS, LANES)
    ssq_p = ssq_p.reshape(NUM_WORKERS, BANKS, BINS, LANES)
    loss = pl.pallas_call(
        _tc_finish_body,
        out_shape=jax.ShapeDtypeStruct((1, 1), jnp.float32),
        out_specs=pl.BlockSpec(memory_space=pltpu.SMEM),
    )(cnt_p, ssq_p)
    return loss[0, 0]


# speedup vs baseline: 1.6093x; 1.0174x over previous
import functools

import jax
import jax.numpy as jnp
from jax import lax
from jax.experimental import pallas as pl
from jax.experimental.pallas import tpu as pltpu
from jax.experimental.pallas import tpu_sc as plsc

N = 1048576
NUM_CORES = 2
NUM_SUBCORES = 16
NUM_WORKERS = NUM_CORES * NUM_SUBCORES
CHUNK = N // NUM_WORKERS
LANES = 16
BINS = 64
OFFSET = 32
BANKS = 4
BANK_WORDS = BINS * LANES
HIST_WORDS = BANKS * BANK_WORDS
ROWS = BANKS * LANES
ITERS = CHUNK // (LANES * BANKS)
MAGIC = 12582912.0


def _sc_hist_body(t_hbm, o_hbm, cnt_out, ssq_out, t_v, o_v, cnt_v, ssq_v,
                  cnt_tot, ssq_tot, sem_t, sem_o):
    wid = lax.axis_index("s") * NUM_CORES + lax.axis_index("c")
    base = wid * CHUNK

    cp_t = pltpu.make_async_copy(t_hbm.at[pl.ds(base, CHUNK)], t_v, sem_t)
    cp_o = pltpu.make_async_copy(o_hbm.at[pl.ds(base, CHUNK)], o_v, sem_o)
    cp_t.start()
    cp_o.start()

    zeros = jnp.zeros((LANES,), jnp.float32)
    for j in range(HIST_WORDS // LANES):
        cnt_v[pl.ds(j * LANES, LANES)] = zeros
        ssq_v[pl.ds(j * LANES, LANES)] = zeros

    lane = lax.iota(jnp.int32, LANES)
    ones = jnp.ones((LANES,), jnp.float32)
    lane_base = lane * jnp.int32(BINS)
    addc = [(lane_base + jnp.int32(k * BANK_WORDS + OFFSET)).astype(jnp.float32)
            for k in range(BANKS)]
    lo = [lane_base + jnp.int32(k * BANK_WORDS) for k in range(BANKS)]
    hi = [lane_base + jnp.int32(k * BANK_WORDS + BINS - 1) for k in range(BANKS)]

    cp_t.wait()
    cp_o.wait()

    @plsc.parallel_loop(0, ITERS, 1, unroll=2)
    def body(i):
        off = i * (LANES * BANKS)
        for k in range(BANKS):
            s = off + k * LANES
            t = t_v[pl.ds(s, LANES)]
            o = o_v[pl.ds(s, LANES)]
            d = t - o
            d2 = d * d
            r = (t + MAGIC) - MAGIC
            flat = (r + addc[k]).astype(jnp.int32)
            flat = jnp.minimum(jnp.maximum(flat, lo[k]), hi[k])
            plsc.addupdate_scatter(cnt_v, [flat], ones)
            plsc.addupdate_scatter(ssq_v, [flat], d2)

    for c in range(BINS // LANES):
        ccol = c * LANES

        @plsc.parallel_loop(0, ROWS, 1, unroll=4,
                            carry=(jnp.zeros((LANES,), jnp.float32),
                                   jnp.zeros((LANES,), jnp.float32)))
        def merge(row, acc):
            a_c, a_s = acc
            a_c = a_c + cnt_v[pl.ds(row * BINS + ccol, LANES)]
            a_s = a_s + ssq_v[pl.ds(row * BINS + ccol, LANES)]
            return a_c, a_s

        tot_c, tot_s = merge
        cnt_tot[pl.ds(ccol, LANES)] = tot_c
        ssq_tot[pl.ds(ccol, LANES)] = tot_s

    pltpu.sync_copy(cnt_tot, cnt_out.at[wid])
    pltpu.sync_copy(ssq_tot, ssq_out.at[wid])


_sc_hist = functools.partial(
    pl.kernel,
    out_type=[
        jax.ShapeDtypeStruct((NUM_WORKERS, BINS), jnp.float32),
        jax.ShapeDtypeStruct((NUM_WORKERS, BINS), jnp.float32),
    ],
    mesh=plsc.VectorSubcoreMesh(core_axis_name="c", subcore_axis_name="s"),
    compiler_params=pltpu.CompilerParams(needs_layout_passes=False),
    scratch_types=[
        pltpu.VMEM((CHUNK,), jnp.float32),
        pltpu.VMEM((CHUNK,), jnp.float32),
        pltpu.VMEM((HIST_WORDS,), jnp.float32),
        pltpu.VMEM((HIST_WORDS,), jnp.float32),
        pltpu.VMEM((BINS,), jnp.float32),
        pltpu.VMEM((BINS,), jnp.float32),
        pltpu.SemaphoreType.DMA,
        pltpu.SemaphoreType.DMA,
    ],
)(_sc_hist_body)


def _tc_finish_body(cnt_ref, ssq_ref, out_ref):
    cnt = jnp.sum(cnt_ref[...], axis=0)
    ssq = jnp.sum(ssq_ref[...], axis=0)
    maxw = jnp.max(cnt)
    nonzero = cnt > 0.0
    safe = jnp.where(nonzero, cnt, 1.0)
    total = jnp.sum(jnp.where(nonzero, ssq / safe, 0.0))
    out_ref[0, 0] = maxw * total * (1.0 / N)


def kernel(target, output):
    t = target.reshape(N)
    o = output.reshape(N)
    cnt_p, ssq_p = _sc_hist(t, o)
    loss = pl.pallas_call(
        _tc_finish_body,
        out_shape=jax.ShapeDtypeStruct((1, 1), jnp.float32),
        out_specs=pl.BlockSpec(memory_space=pltpu.SMEM),
    )(cnt_p, ssq_p)
    return loss[0, 0]


# speedup vs baseline: 2.1204x; 1.3176x over previous
import functools

import jax
import jax.numpy as jnp
from jax import lax
from jax.experimental import pallas as pl
from jax.experimental.pallas import tpu as pltpu
from jax.experimental.pallas import tpu_sc as plsc

N = 1048576
NUM_CORES = 2
NUM_SUBCORES = 16
NUM_WORKERS = NUM_CORES * NUM_SUBCORES
CHUNK = N // NUM_WORKERS
LANES = 16
BINS = 64
OFFSET = 32
BANKS = 4
BANK_WORDS = BINS * LANES
HIST_WORDS = BANKS * BANK_WORDS
ROWS = BANKS * LANES
ITERS = CHUNK // (LANES * BANKS)
MAGIC = 12582912.0


def _sc_hist_body(t_hbm, o_hbm, cnt_out, ssq_out, t_v, o_v, cnt_v, ssq_v,
                  cnt_tot, ssq_tot, sem_t, sem_o):
    wid = lax.axis_index("s") * NUM_CORES + lax.axis_index("c")
    base = wid * CHUNK

    cp_t = pltpu.make_async_copy(t_hbm.at[pl.ds(base, CHUNK)], t_v, sem_t)
    cp_o = pltpu.make_async_copy(o_hbm.at[pl.ds(base, CHUNK)], o_v, sem_o)
    cp_t.start()
    cp_o.start()

    zeros = jnp.zeros((LANES,), jnp.float32)
    for j in range(HIST_WORDS // LANES):
        cnt_v[pl.ds(j * LANES, LANES)] = zeros
        ssq_v[pl.ds(j * LANES, LANES)] = zeros

    lane = lax.iota(jnp.int32, LANES)
    ones = jnp.ones((LANES,), jnp.float32)
    lane_f = lane.astype(jnp.float32)
    addc = [lane_f + float(k * BANK_WORDS + OFFSET * LANES) for k in range(BANKS)]
    lo = [lane + jnp.int32(k * BANK_WORDS) for k in range(BANKS)]
    hi = [lane + jnp.int32(k * BANK_WORDS + (BINS - 1) * LANES) for k in range(BANKS)]

    cp_t.wait()
    cp_o.wait()

    @plsc.parallel_loop(0, ITERS, 1, unroll=2)
    def body(i):
        off = i * (LANES * BANKS)
        for k in range(BANKS):
            s = off + k * LANES
            t = t_v[pl.ds(s, LANES)]
            o = o_v[pl.ds(s, LANES)]
            d = t - o
            d2 = d * d
            r = (t + MAGIC) - MAGIC
            flat = (r * float(LANES) + addc[k]).astype(jnp.int32)
            flat = jnp.minimum(jnp.maximum(flat, lo[k]), hi[k])
            plsc.addupdate_scatter(cnt_v, [flat], ones)
            plsc.addupdate_scatter(ssq_v, [flat], d2)

    for c in range(BINS // LANES):
        acc_c = zeros
        acc_s = zeros
        for i in range(LANES):
            w = (c * LANES + i) * LANES
            vc = cnt_v[pl.ds(w, LANES)]
            vs = ssq_v[pl.ds(w, LANES)]
            for k in range(1, BANKS):
                vc = vc + cnt_v[pl.ds(k * BANK_WORDS + w, LANES)]
                vs = vs + ssq_v[pl.ds(k * BANK_WORDS + w, LANES)]
            acc_c = jnp.where(lane == i, jnp.sum(vc), acc_c)
            acc_s = jnp.where(lane == i, jnp.sum(vs), acc_s)
        cnt_tot[pl.ds(c * LANES, LANES)] = acc_c
        ssq_tot[pl.ds(c * LANES, LANES)] = acc_s

    pltpu.sync_copy(cnt_tot, cnt_out.at[wid])
    pltpu.sync_copy(ssq_tot, ssq_out.at[wid])


_sc_hist = functools.partial(
    pl.kernel,
    out_type=[
        jax.ShapeDtypeStruct((NUM_WORKERS, BINS), jnp.float32),
        jax.ShapeDtypeStruct((NUM_WORKERS, BINS), jnp.float32),
    ],
    mesh=plsc.VectorSubcoreMesh(core_axis_name="c", subcore_axis_name="s"),
    compiler_params=pltpu.CompilerParams(needs_layout_passes=False),
    scratch_types=[
        pltpu.VMEM((CHUNK,), jnp.float32),
        pltpu.VMEM((CHUNK,), jnp.float32),
        pltpu.VMEM((HIST_WORDS,), jnp.float32),
        pltpu.VMEM((HIST_WORDS,), jnp.float32),
        pltpu.VMEM((BINS,), jnp.float32),
        pltpu.VMEM((BINS,), jnp.float32),
        pltpu.SemaphoreType.DMA,
        pltpu.SemaphoreType.DMA,
    ],
)(_sc_hist_body)


def _tc_finish_body(cnt_ref, ssq_ref, out_ref):
    cnt = jnp.sum(cnt_ref[...], axis=0)
    ssq = jnp.sum(ssq_ref[...], axis=0)
    maxw = jnp.max(cnt)
    nonzero = cnt > 0.0
    safe = jnp.where(nonzero, cnt, 1.0)
    total = jnp.sum(jnp.where(nonzero, ssq / safe, 0.0))
    out_ref[0, 0] = maxw * total * (1.0 / N)


def kernel(target, output):
    t = target.reshape(N)
    o = output.reshape(N)
    cnt_p, ssq_p = _sc_hist(t, o)
    loss = pl.pallas_call(
        _tc_finish_body,
        out_shape=jax.ShapeDtypeStruct((1, 1), jnp.float32),
        out_specs=pl.BlockSpec(memory_space=pltpu.SMEM),
    )(cnt_p, ssq_p)
    return loss[0, 0]
